# baseline (speedup 1.0000x reference)
import functools

import jax
import jax.numpy as jnp
from jax import lax
from jax.experimental import pallas as pl
from jax.experimental.pallas import tpu as pltpu
from jax.experimental.pallas import tpu_sc as plsc

NUM_FACTOR = 64
BATCH = 16384

_NC = 2
_NS = 16
_NW = _NC * _NS
_ROWS_PER_W = BATCH // _NW
_C = 128
_N_CHUNKS = _ROWS_PER_W // _C
_L = 16
_PAIR = 2 * NUM_FACTOR

_GATHER_DNUMS = lax.GatherDimensionNumbers(
    offset_dims=(), collapsed_slice_dims=(0,), start_index_map=(0,))


def _rot_gather(v, idx):
    return lax.gather(v, idx[:, None], _GATHER_DNUMS, slice_sizes=(1,),
                      mode=lax.GatherScatterMode.PROMISE_IN_BOUNDS)


def _gmf_body(uidx_hbm, iidx_hbm, utab_hbm, itab_hbm, w_hbm, out_hbm,
              uraw_v, iraw_v, utix_v, itix_v, ubuf, ibuf, w_v, out_v, sem):
    wid = lax.axis_index("s") * _NC + lax.axis_index("c")
    base = wid * _ROWS_PER_W

    pltpu.sync_copy(uidx_hbm.at[pl.ds(base, _ROWS_PER_W)], uraw_v)
    pltpu.sync_copy(iidx_hbm.at[pl.ds(base, _ROWS_PER_W)], iraw_v)
    pltpu.sync_copy(w_hbm, w_v)

    def tix(k, carry):
        utix_v[pl.ds(k * _L, _L)] = lax.shift_right_logical(
            uraw_v[pl.ds(k * _L, _L)], 1)
        itix_v[pl.ds(k * _L, _L)] = lax.shift_right_logical(
            iraw_v[pl.ds(k * _L, _L)], 1)
        return carry

    lax.fori_loop(0, _ROWS_PER_W // _L, tix, 0)

    def fire_chunk(c, slot):
        pltpu.async_copy(
            utab_hbm.at[utix_v.at[pl.ds(c * _C, _C)]], ubuf.at[slot], sem)
        pltpu.async_copy(
            itab_hbm.at[itix_v.at[pl.ds(c * _C, _C)]], ibuf.at[slot], sem)

    def drain_chunk(slot):
        pltpu.make_async_copy(
            utab_hbm.at[utix_v.at[pl.ds(0, _C)]], ubuf.at[slot], sem).wait()
        pltpu.make_async_copy(
            itab_hbm.at[itix_v.at[pl.ds(0, _C)]], ibuf.at[slot], sem).wait()

    w0 = w_v[pl.ds(0, _L)]
    w1 = w_v[pl.ds(_L, _L)]
    w2 = w_v[pl.ds(2 * _L, _L)]
    w3 = w_v[pl.ds(3 * _L, _L)]
    lane_ids = lax.iota(jnp.int32, _L)
    onehot = [lane_ids == l for l in range(_L)]
    rot_idx = [(lane_ids + sh) & (_L - 1) for sh in (8, 4, 2, 1)]

    def compute_chunk(c, slot):
        def block(b, carry2):
            bbase = b * _L
            su_vec = (uraw_v[pl.ds(c * _C + bbase, _L)] & 1) * NUM_FACTOR
            si_vec = (iraw_v[pl.ds(c * _C + bbase, _L)] & 1) * NUM_FACTOR
            acc = jnp.zeros((_L,), jnp.float32)
            for l in range(_L):
                j = bbase + l
                su = su_vec[l]
                si = si_vec[l]
                v = (ubuf[slot, j, pl.ds(su, _L)] * ibuf[slot, j, pl.ds(si, _L)] * w0
                     + ubuf[slot, j, pl.ds(su + _L, _L)] * ibuf[slot, j, pl.ds(si + _L, _L)] * w1
                     + ubuf[slot, j, pl.ds(su + 2 * _L, _L)] * ibuf[slot, j, pl.ds(si + 2 * _L, _L)] * w2
                     + ubuf[slot, j, pl.ds(su + 3 * _L, _L)] * ibuf[slot, j, pl.ds(si + 3 * _L, _L)] * w3)
                for idx in rot_idx:
                    v = v + _rot_gather(v, idx)
                acc = jnp.where(onehot[l], v, acc)
            out_v[pl.ds(c * _C + bbase, _L)] = acc
            return carry2

        lax.fori_loop(0, _C // _L, block, 0)

    fire_chunk(0, 0)

    def step(c, carry):
        slot = lax.rem(c, 2)
        nslot = lax.rem(c + 1, 2)

        @pl.when(c + 1 < _N_CHUNKS)
        def _():
            fire_chunk(c + 1, nslot)

        drain_chunk(slot)
        compute_chunk(c, slot)
        return carry

    lax.fori_loop(0, _N_CHUNKS, step, 0)

    pltpu.sync_copy(out_v, out_hbm.at[pl.ds(base, _ROWS_PER_W)])


@jax.jit
def _gmf(user, item, user_table, item_table, w_flat):
    utab = user_table.reshape(-1, _PAIR)
    itab = item_table.reshape(-1, _PAIR)
    mesh = plsc.VectorSubcoreMesh(core_axis_name="c", subcore_axis_name="s")
    run = functools.partial(
        pl.kernel, mesh=mesh,
        out_type=jax.ShapeDtypeStruct((BATCH,), jnp.float32),
        scratch_types=[
            pltpu.VMEM((_ROWS_PER_W,), jnp.int32),
            pltpu.VMEM((_ROWS_PER_W,), jnp.int32),
            pltpu.VMEM((_ROWS_PER_W,), jnp.int32),
            pltpu.VMEM((_ROWS_PER_W,), jnp.int32),
            pltpu.VMEM((2, _C, _PAIR), jnp.float32),
            pltpu.VMEM((2, _C, _PAIR), jnp.float32),
            pltpu.VMEM((NUM_FACTOR,), jnp.float32),
            pltpu.VMEM((_ROWS_PER_W,), jnp.float32),
            pltpu.SemaphoreType.DMA,
        ],
    )(_gmf_body)
    return run(user, item, utab, itab, w_flat)


def kernel(user, item, user_table, item_table, W):
    return _gmf(user, item, user_table, item_table, W.reshape(-1))


# speedup vs baseline: 3.1044x; 3.1044x over previous
import functools

import jax
import jax.numpy as jnp
from jax import lax
from jax.experimental import pallas as pl
from jax.experimental.pallas import tpu as pltpu
from jax.experimental.pallas import tpu_sc as plsc

NUM_FACTOR = 64
BATCH = 16384

_NC = 2
_NS = 16
_NW = _NC * _NS
_RPW = BATCH // _NW
_L = 16
_WINW = 512
_NROW = 1000000
_TAIL_OFF = (_NROW // 128) * 128
_TAILW = _NROW - _TAIL_OFF
_MAX_WOFF = _TAIL_OFF - _WINW

_GATHER_DNUMS = lax.GatherDimensionNumbers(
    offset_dims=(), collapsed_slice_dims=(0,), start_index_map=(0,))


def _rot_gather(v, idx):
    return lax.gather(v, idx[:, None], _GATHER_DNUMS, slice_sizes=(1,),
                      mode=lax.GatherScatterMode.PROMISE_IN_BOUNDS)


def _win_off(t, base_off):
    return jnp.minimum(base_off + t * _WINW, _MAX_WOFF)


def _tail_fmt_body(ttu_hbm, tti_hbm, outu_hbm, outi_hbm, buf, outv, sem):
    wid = lax.axis_index("s") * _NC + lax.axis_index("c")

    def emit(tt_hbm, out_hbm):
        pltpu.sync_copy(tt_hbm.at[:, pl.ds(_TAIL_OFF, _TAILW)], buf)
        for f in range(NUM_FACTOR):
            for c in range(_TAILW // _L):
                outv[pl.ds(f * _TAILW + c * _L, _L)] = buf[f, pl.ds(c * _L, _L)]
        pltpu.sync_copy(outv, out_hbm)

    @pl.when(wid == 0)
    def _():
        emit(ttu_hbm, outu_hbm)

    @pl.when(wid == 1)
    def _():
        emit(tti_hbm, outi_hbm)


def _extract_body(ids_hbm, pos_hbm, tt_hbm, tlin_hbm, emb_hbm,
                  ids_v, pos_v, chunk, tail, stage, sem_c, sem_r):
    wid = lax.axis_index("s") * _NC + lax.axis_index("c")
    base = wid * _RPW

    pltpu.sync_copy(ids_hbm.at[pl.ds(base, _RPW)], ids_v)
    pltpu.sync_copy(pos_hbm.at[pl.ds(base, _RPW)], pos_v)
    pltpu.sync_copy(tlin_hbm, tail)

    base_off = jnp.minimum(
        lax.shift_right_logical(ids_v[pl.ds(0, _L)][0], 7) * 128, _MAX_WOFF)
    t_cap = (_MAX_WOFF - base_off + _WINW - 1) // _WINW

    def fire(t, slot):
        woff = pl.multiple_of(_win_off(t, base_off), 128)

        def body(fb, carry):
            pltpu.async_copy(
                tt_hbm.at[pl.ds(fb * 8, 8), pl.ds(woff, _WINW)],
                chunk.at[slot, fb], sem_c)
            return carry

        lax.fori_loop(0, 8, body, 0)

    def drain_win():
        def body(fb, carry):
            pltpu.make_async_copy(
                tt_hbm.at[pl.ds(0, 8), pl.ds(0, _WINW)],
                chunk.at[0, 0], sem_c).wait()
            return carry

        lax.fori_loop(0, 8, body, 0)

    fire(0, 0)
    fire(1, 1)
    drain_win()

    lane = lax.iota(jnp.int32, _L)

    def advance(cb_l, t):
        t_tgt = jnp.maximum(
            t, jnp.minimum((cb_l * 128 - base_off) // _WINW, t_cap))

        def body(s, carry):
            drain_win()
            fire(s + 2, lax.rem(s, 2))
            return carry

        lax.fori_loop(t, t_tgt, body, 0)
        return t_tgt

    def block(b, t):
        idvec = ids_v[pl.ds(b * _L, _L)]
        posvec = pos_v[pl.ds(b * _L, _L)]
        cbvec = lax.shift_right_logical(idvec, 7)
        for l in range(_L):
            t = advance(cbvec[l], t)
            slot = lax.rem(t, 2)
            id_l = idvec[l]
            col = jnp.clip(id_l - _win_off(t, base_off), 0, _WINW - 1)
            cbl = lax.shift_right_logical(col, 7)
            lcol = col & 127
            sr = l & 7
            if l >= 8:
                pltpu.make_async_copy(
                    emb_hbm.at[pl.ds(0, NUM_FACTOR)],
                    stage.at[pl.ds(0, NUM_FACTOR)], sem_r).wait()
            else:
                @pl.when(b > 0)
                def _():
                    pltpu.make_async_copy(
                        emb_hbm.at[pl.ds(0, NUM_FACTOR)],
                        stage.at[pl.ds(0, NUM_FACTOR)], sem_r).wait()
            for fbg in range(4):
                f = fbg * _L + lane
                i1 = lax.shift_right_logical(f, 3)
                i2 = f & 7
                i3 = jnp.full((_L,), col, jnp.int32)
                vals = plsc.load_gather(
                    chunk, [jnp.full((_L,), slot, jnp.int32), i1, i2, i3])
                stage[pl.ds(sr * NUM_FACTOR + fbg * _L, _L)] = vals

            @pl.when(id_l >= _TAIL_OFF)
            def _():
                tcol = id_l - _TAIL_OFF
                for fbg in range(4):
                    tidx = (fbg * _L + lane) * _TAILW + tcol
                    tvals = plsc.load_gather(tail, [tidx])
                    stage[pl.ds(sr * NUM_FACTOR + fbg * _L, _L)] = tvals

            pltpu.async_copy(
                stage.at[pl.ds(sr * NUM_FACTOR, NUM_FACTOR)],
                emb_hbm.at[pl.ds(posvec[l] * NUM_FACTOR, NUM_FACTOR)],
                sem_r)
        return t

    lax.fori_loop(0, _RPW // _L, block, 0)
    drain_win()
    for _ in range(8):
        pltpu.make_async_copy(
            emb_hbm.at[pl.ds(0, NUM_FACTOR)],
            stage.at[pl.ds(0, NUM_FACTOR)], sem_r).wait()


def _combine_body(uemb_hbm, iemb_hbm, w_hbm, out_hbm,
                  u_v, i_v, w_v, out_v, sem):
    wid = lax.axis_index("s") * _NC + lax.axis_index("c")
    base = wid * _RPW * NUM_FACTOR

    pltpu.sync_copy(uemb_hbm.at[pl.ds(base, _RPW * NUM_FACTOR)], u_v)
    pltpu.sync_copy(iemb_hbm.at[pl.ds(base, _RPW * NUM_FACTOR)], i_v)
    pltpu.sync_copy(w_hbm, w_v)

    w0 = w_v[pl.ds(0, _L)]
    w1 = w_v[pl.ds(_L, _L)]
    w2 = w_v[pl.ds(2 * _L, _L)]
    w3 = w_v[pl.ds(3 * _L, _L)]
    lane_ids = lax.iota(jnp.int32, _L)
    onehot = [lane_ids == l for l in range(_L)]
    rot_idx = [(lane_ids + sh) & (_L - 1) for sh in (8, 4, 2, 1)]

    def blk(b, carry):
        acc = jnp.zeros((_L,), jnp.float32)
        for l in range(_L):
            o = (b * _L + l) * NUM_FACTOR
            v = (u_v[pl.ds(o, _L)] * i_v[pl.ds(o, _L)] * w0
                 + u_v[pl.ds(o + _L, _L)] * i_v[pl.ds(o + _L, _L)] * w1
                 + u_v[pl.ds(o + 2 * _L, _L)] * i_v[pl.ds(o + 2 * _L, _L)] * w2
                 + u_v[pl.ds(o + 3 * _L, _L)] * i_v[pl.ds(o + 3 * _L, _L)] * w3)
            for idx in rot_idx:
                v = v + _rot_gather(v, idx)
            acc = jnp.where(onehot[l], v, acc)
        out_v[pl.ds(b * _L, _L)] = acc
        return carry

    lax.fori_loop(0, _RPW // _L, blk, 0)
    pltpu.sync_copy(out_v, out_hbm.at[pl.ds(wid * _RPW, _RPW)])


@jax.jit
def _gmf(user, item, user_table, item_table, w_flat):
    mesh = plsc.VectorSubcoreMesh(core_axis_name="c", subcore_axis_name="s")
    iota_b = lax.iota(jnp.int32, BATCH)
    us, upos = lax.sort_key_val(user, iota_b)
    its, ipos = lax.sort_key_val(item, iota_b)
    ttu = user_table.T
    tti = item_table.T

    tail_fmt = functools.partial(
        pl.kernel, mesh=mesh,
        out_type=(jax.ShapeDtypeStruct((NUM_FACTOR * _TAILW,), jnp.float32),
                  jax.ShapeDtypeStruct((NUM_FACTOR * _TAILW,), jnp.float32)),
        scratch_types=[
            pltpu.VMEM((NUM_FACTOR, _TAILW), jnp.float32),
            pltpu.VMEM((NUM_FACTOR * _TAILW,), jnp.float32),
            pltpu.SemaphoreType.DMA,
        ],
    )(_tail_fmt_body)
    tlu, tli = tail_fmt(ttu, tti)

    extract = functools.partial(
        pl.kernel, mesh=mesh,
        compiler_params=pltpu.CompilerParams(needs_layout_passes=False),
        out_type=jax.ShapeDtypeStruct((BATCH * NUM_FACTOR,), jnp.float32),
        scratch_types=[
            pltpu.VMEM((_RPW,), jnp.int32),
            pltpu.VMEM((_RPW,), jnp.int32),
            pltpu.VMEM((2, 8, 8, _WINW), jnp.float32),
            pltpu.VMEM((NUM_FACTOR * _TAILW,), jnp.float32),
            pltpu.VMEM((8 * NUM_FACTOR,), jnp.float32),
            pltpu.SemaphoreType.DMA,
            pltpu.SemaphoreType.DMA,
        ],
    )(_extract_body)
    uemb = extract(us, upos, ttu, tlu)
    iemb = extract(its, ipos, tti, tli)

    combine = functools.partial(
        pl.kernel, mesh=mesh,
        out_type=jax.ShapeDtypeStruct((BATCH,), jnp.float32),
        scratch_types=[
            pltpu.VMEM((_RPW * NUM_FACTOR,), jnp.float32),
            pltpu.VMEM((_RPW * NUM_FACTOR,), jnp.float32),
            pltpu.VMEM((NUM_FACTOR,), jnp.float32),
            pltpu.VMEM((_RPW,), jnp.float32),
            pltpu.SemaphoreType.DMA,
        ],
    )(_combine_body)
    return combine(uemb, iemb, w_flat)


def kernel(user, item, user_table, item_table, W):
    return _gmf(user, item, user_table, item_table, W.reshape(-1))


# speedup vs baseline: 3.9239x; 1.2640x over previous
import functools

import jax
import jax.numpy as jnp
from jax import lax
from jax.experimental import pallas as pl
from jax.experimental.pallas import tpu as pltpu
from jax.experimental.pallas import tpu_sc as plsc

NUM_FACTOR = 64
BATCH = 16384

_NC = 2
_NS = 16
_NW = _NC * _NS
_RPW = BATCH // _NW
_L = 16
_WINW = 512
_NROW = 1000000
_TAIL_OFF = (_NROW // 128) * 128
_TAILW = _NROW - _TAIL_OFF
_MAX_WOFF = _TAIL_OFF - _WINW

_GATHER_DNUMS = lax.GatherDimensionNumbers(
    offset_dims=(), collapsed_slice_dims=(0,), start_index_map=(0,))


def _rot_gather(v, idx):
    return lax.gather(v, idx[:, None], _GATHER_DNUMS, slice_sizes=(1,),
                      mode=lax.GatherScatterMode.PROMISE_IN_BOUNDS)


def _win_off(t, base_off):
    return jnp.minimum(base_off + t * _WINW, _MAX_WOFF)


def _tail_fmt_body(ttu_hbm, tti_hbm, outu_hbm, outi_hbm, buf, outv, sem):
    wid = lax.axis_index("s") * _NC + lax.axis_index("c")

    def emit(tt_hbm, out_hbm):
        pltpu.sync_copy(tt_hbm.at[:, pl.ds(_TAIL_OFF, _TAILW)], buf)
        for f in range(NUM_FACTOR):
            for c in range(_TAILW // _L):
                outv[pl.ds(f * _TAILW + c * _L, _L)] = buf[f, pl.ds(c * _L, _L)]
        pltpu.sync_copy(outv, out_hbm)

    @pl.when(wid == 0)
    def _():
        emit(ttu_hbm, outu_hbm)

    @pl.when(wid == 1)
    def _():
        emit(tti_hbm, outi_hbm)


def _extract_body(ids_hbm, pos_hbm, tt_hbm, tlin_hbm, emb_hbm,
                  ids_v, pos_v, chunk, tail, stage, sem_c, sem_r):
    wid = lax.axis_index("s") * _NC + lax.axis_index("c")
    base = wid * _RPW

    pltpu.sync_copy(ids_hbm.at[pl.ds(base, _RPW)], ids_v)
    pltpu.sync_copy(pos_hbm.at[pl.ds(base, _RPW)], pos_v)
    pltpu.sync_copy(tlin_hbm, tail)

    base_off = jnp.minimum(
        lax.shift_right_logical(ids_v[pl.ds(0, _L)][0], 7) * 128, _MAX_WOFF)
    t_cap = (_MAX_WOFF - base_off + _WINW - 1) // _WINW

    def fire(t, slot):
        woff = pl.multiple_of(_win_off(t, base_off), 128)

        def body(fb, carry):
            pltpu.async_copy(
                tt_hbm.at[pl.ds(fb * 8, 8), pl.ds(woff, _WINW)],
                chunk.at[slot, fb], sem_c)
            return carry

        lax.fori_loop(0, 8, body, 0)

    def drain_win():
        def body(fb, carry):
            pltpu.make_async_copy(
                tt_hbm.at[pl.ds(0, 8), pl.ds(0, _WINW)],
                chunk.at[0, 0], sem_c).wait()
            return carry

        lax.fori_loop(0, 8, body, 0)

    fire(0, 0)
    fire(1, 1)
    fire(2, 2)
    drain_win()

    lane = lax.iota(jnp.int32, _L)

    def advance(cb_l, t):
        t_tgt = jnp.maximum(
            t, jnp.minimum((cb_l * 128 - base_off) // _WINW, t_cap))

        def body(s, carry):
            drain_win()
            fire(s + 3, lax.rem(s + 3, 3))
            return carry

        lax.fori_loop(t, t_tgt, body, 0)
        return t_tgt

    def block(b, t):
        idvec = ids_v[pl.ds(b * _L, _L)]
        posvec = pos_v[pl.ds(b * _L, _L)]
        cbvec = lax.shift_right_logical(idvec, 7)
        for l in range(_L):
            t = advance(cbvec[l], t)
            slot = lax.rem(t, 3)
            id_l = idvec[l]
            col = jnp.clip(id_l - _win_off(t, base_off), 0, _WINW - 1)
            cbl = lax.shift_right_logical(col, 7)
            lcol = col & 127
            sr = l & 7
            if l >= 8:
                pltpu.make_async_copy(
                    emb_hbm.at[pl.ds(0, NUM_FACTOR)],
                    stage.at[pl.ds(0, NUM_FACTOR)], sem_r).wait()
            else:
                @pl.when(b > 0)
                def _():
                    pltpu.make_async_copy(
                        emb_hbm.at[pl.ds(0, NUM_FACTOR)],
                        stage.at[pl.ds(0, NUM_FACTOR)], sem_r).wait()
            for fbg in range(4):
                f = fbg * _L + lane
                i1 = lax.shift_right_logical(f, 3)
                i2 = f & 7
                i3 = jnp.full((_L,), col, jnp.int32)
                vals = plsc.load_gather(
                    chunk, [jnp.full((_L,), slot, jnp.int32), i1, i2, i3])
                stage[pl.ds(sr * NUM_FACTOR + fbg * _L, _L)] = vals

            @pl.when(id_l >= _TAIL_OFF)
            def _():
                tcol = id_l - _TAIL_OFF
                for fbg in range(4):
                    tidx = (fbg * _L + lane) * _TAILW + tcol
                    tvals = plsc.load_gather(tail, [tidx])
                    stage[pl.ds(sr * NUM_FACTOR + fbg * _L, _L)] = tvals

            pltpu.async_copy(
                stage.at[pl.ds(sr * NUM_FACTOR, NUM_FACTOR)],
                emb_hbm.at[pl.ds(posvec[l] * NUM_FACTOR, NUM_FACTOR)],
                sem_r)
        return t

    lax.fori_loop(0, _RPW // _L, block, 0)
    drain_win()
    drain_win()
    for _ in range(8):
        pltpu.make_async_copy(
            emb_hbm.at[pl.ds(0, NUM_FACTOR)],
            stage.at[pl.ds(0, NUM_FACTOR)], sem_r).wait()


def _combine_body(uemb_hbm, iemb_hbm, w_hbm, out_hbm,
                  u_v, i_v, w_v, out_v, sem):
    wid = lax.axis_index("s") * _NC + lax.axis_index("c")
    base = wid * _RPW * NUM_FACTOR

    pltpu.sync_copy(uemb_hbm.at[pl.ds(base, _RPW * NUM_FACTOR)], u_v)
    pltpu.sync_copy(iemb_hbm.at[pl.ds(base, _RPW * NUM_FACTOR)], i_v)
    pltpu.sync_copy(w_hbm, w_v)

    w0 = w_v[pl.ds(0, _L)]
    w1 = w_v[pl.ds(_L, _L)]
    w2 = w_v[pl.ds(2 * _L, _L)]
    w3 = w_v[pl.ds(3 * _L, _L)]
    lane_ids = lax.iota(jnp.int32, _L)
    onehot = [lane_ids == l for l in range(_L)]
    rot_idx = [(lane_ids + sh) & (_L - 1) for sh in (8, 4, 2, 1)]

    def blk(b, carry):
        acc = jnp.zeros((_L,), jnp.float32)
        for l in range(_L):
            o = (b * _L + l) * NUM_FACTOR
            v = (u_v[pl.ds(o, _L)] * i_v[pl.ds(o, _L)] * w0
                 + u_v[pl.ds(o + _L, _L)] * i_v[pl.ds(o + _L, _L)] * w1
                 + u_v[pl.ds(o + 2 * _L, _L)] * i_v[pl.ds(o + 2 * _L, _L)] * w2
                 + u_v[pl.ds(o + 3 * _L, _L)] * i_v[pl.ds(o + 3 * _L, _L)] * w3)
            for idx in rot_idx:
                v = v + _rot_gather(v, idx)
            acc = jnp.where(onehot[l], v, acc)
        out_v[pl.ds(b * _L, _L)] = acc
        return carry

    lax.fori_loop(0, _RPW // _L, blk, 0)
    pltpu.sync_copy(out_v, out_hbm.at[pl.ds(wid * _RPW, _RPW)])


@jax.jit
def _gmf(user, item, user_table, item_table, w_flat):
    mesh = plsc.VectorSubcoreMesh(core_axis_name="c", subcore_axis_name="s")
    iota_b = lax.iota(jnp.int32, BATCH)
    us, upos = lax.sort_key_val(user, iota_b)
    its, ipos = lax.sort_key_val(item, iota_b)
    ttu = user_table.T
    tti = item_table.T

    tail_fmt = functools.partial(
        pl.kernel, mesh=mesh,
        out_type=(jax.ShapeDtypeStruct((NUM_FACTOR * _TAILW,), jnp.float32),
                  jax.ShapeDtypeStruct((NUM_FACTOR * _TAILW,), jnp.float32)),
        scratch_types=[
            pltpu.VMEM((NUM_FACTOR, _TAILW), jnp.float32),
            pltpu.VMEM((NUM_FACTOR * _TAILW,), jnp.float32),
            pltpu.SemaphoreType.DMA,
        ],
    )(_tail_fmt_body)
    tlu, tli = tail_fmt(ttu, tti)

    extract = functools.partial(
        pl.kernel, mesh=mesh,
        compiler_params=pltpu.CompilerParams(needs_layout_passes=False),
        out_type=jax.ShapeDtypeStruct((BATCH * NUM_FACTOR,), jnp.float32),
        scratch_types=[
            pltpu.VMEM((_RPW,), jnp.int32),
            pltpu.VMEM((_RPW,), jnp.int32),
            pltpu.VMEM((3, 8, 8, _WINW), jnp.float32),
            pltpu.VMEM((NUM_FACTOR * _TAILW,), jnp.float32),
            pltpu.VMEM((8 * NUM_FACTOR,), jnp.float32),
            pltpu.SemaphoreType.DMA,
            pltpu.SemaphoreType.DMA,
        ],
    )(_extract_body)
    uemb = extract(us, upos, ttu, tlu)
    iemb = extract(its, ipos, tti, tli)

    combine = functools.partial(
        pl.kernel, mesh=mesh,
        out_type=jax.ShapeDtypeStruct((BATCH,), jnp.float32),
        scratch_types=[
            pltpu.VMEM((_RPW * NUM_FACTOR,), jnp.float32),
            pltpu.VMEM((_RPW * NUM_FACTOR,), jnp.float32),
            pltpu.VMEM((NUM_FACTOR,), jnp.float32),
            pltpu.VMEM((_RPW,), jnp.float32),
            pltpu.SemaphoreType.DMA,
        ],
    )(_combine_body)
    return combine(uemb, iemb, w_flat)


def kernel(user, item, user_table, item_table, W):
    return _gmf(user, item, user_table, item_table, W.reshape(-1))


# speedup vs baseline: 4.1127x; 1.0481x over previous
import functools

import jax
import jax.numpy as jnp
from jax import lax
from jax.experimental import pallas as pl
from jax.experimental.pallas import tpu as pltpu
from jax.experimental.pallas import tpu_sc as plsc

NUM_FACTOR = 64
BATCH = 16384

_NC = 2
_NS = 16
_NW = _NC * _NS
_RPW = BATCH // _NW
_L = 16
_WINW = 256
_NROW = 1000000
_TAIL_OFF = (_NROW // 128) * 128
_TAILW = _NROW - _TAIL_OFF
_MAX_WOFF = _TAIL_OFF - _WINW

_GATHER_DNUMS = lax.GatherDimensionNumbers(
    offset_dims=(), collapsed_slice_dims=(0,), start_index_map=(0,))


def _rot_gather(v, idx):
    return lax.gather(v, idx[:, None], _GATHER_DNUMS, slice_sizes=(1,),
                      mode=lax.GatherScatterMode.PROMISE_IN_BOUNDS)


def _win_off(t, base_off):
    return jnp.minimum(base_off + t * _WINW, _MAX_WOFF)


def _tail_fmt_body(ttu_hbm, tti_hbm, outu_hbm, outi_hbm, buf, outv, sem):
    wid = lax.axis_index("s") * _NC + lax.axis_index("c")

    def emit(tt_hbm, out_hbm):
        pltpu.sync_copy(tt_hbm.at[:, pl.ds(_TAIL_OFF, _TAILW)], buf)
        for f in range(NUM_FACTOR):
            for c in range(_TAILW // _L):
                outv[pl.ds(f * _TAILW + c * _L, _L)] = buf[f, pl.ds(c * _L, _L)]
        pltpu.sync_copy(outv, out_hbm)

    @pl.when(wid == 0)
    def _():
        emit(ttu_hbm, outu_hbm)

    @pl.when(wid == 1)
    def _():
        emit(tti_hbm, outi_hbm)


def _extract_body(ids_hbm, pos_hbm, tt_hbm, tlin_hbm, emb_hbm,
                  ids_v, pos_v, chunk, tail, stage, sem_c, sem_r):
    wid = lax.axis_index("s") * _NC + lax.axis_index("c")
    base = wid * _RPW

    pltpu.sync_copy(ids_hbm.at[pl.ds(base, _RPW)], ids_v)
    pltpu.sync_copy(pos_hbm.at[pl.ds(base, _RPW)], pos_v)
    pltpu.sync_copy(tlin_hbm, tail)

    base_off = jnp.minimum(
        lax.shift_right_logical(ids_v[pl.ds(0, _L)][0], 7) * 128, _MAX_WOFF)
    t_cap = (_MAX_WOFF - base_off + _WINW - 1) // _WINW

    def fire(t, slot):
        woff = pl.multiple_of(_win_off(t, base_off), 128)

        def body(fb, carry):
            pltpu.async_copy(
                tt_hbm.at[pl.ds(fb * 8, 8), pl.ds(woff, _WINW)],
                chunk.at[slot, fb], sem_c)
            return carry

        lax.fori_loop(0, 8, body, 0)

    def drain_win():
        def body(fb, carry):
            pltpu.make_async_copy(
                tt_hbm.at[pl.ds(0, 8), pl.ds(0, _WINW)],
                chunk.at[0, 0], sem_c).wait()
            return carry

        lax.fori_loop(0, 8, body, 0)

    for tw in range(6):
        fire(tw, tw)
    drain_win()

    lane = lax.iota(jnp.int32, _L)

    def advance(cb_l, t):
        t_tgt = jnp.maximum(
            t, jnp.minimum((cb_l * 128 - base_off) // _WINW, t_cap))

        def body(s, carry):
            drain_win()
            fire(s + 6, lax.rem(s + 6, 6))
            return carry

        lax.fori_loop(t, t_tgt, body, 0)
        return t_tgt

    def block(b, t):
        idvec = ids_v[pl.ds(b * _L, _L)]
        posvec = pos_v[pl.ds(b * _L, _L)]
        cbvec = lax.shift_right_logical(idvec, 7)
        for l in range(_L):
            t = advance(cbvec[l], t)
            slot = lax.rem(t, 6)
            id_l = idvec[l]
            col = jnp.clip(id_l - _win_off(t, base_off), 0, _WINW - 1)
            cbl = lax.shift_right_logical(col, 7)
            lcol = col & 127
            sr = l
            @pl.when(b > 0)
            def _():
                pltpu.make_async_copy(
                    emb_hbm.at[pl.ds(0, NUM_FACTOR)],
                    stage.at[pl.ds(0, NUM_FACTOR)], sem_r).wait()
            for fbg in range(4):
                f = fbg * _L + lane
                i1 = lax.shift_right_logical(f, 3)
                i2 = f & 7
                i3 = jnp.full((_L,), col, jnp.int32)
                vals = plsc.load_gather(
                    chunk, [jnp.full((_L,), slot, jnp.int32), i1, i2, i3])
                stage[pl.ds(sr * NUM_FACTOR + fbg * _L, _L)] = vals

            @pl.when(id_l >= _TAIL_OFF)
            def _():
                tcol = id_l - _TAIL_OFF
                for fbg in range(4):
                    tidx = (fbg * _L + lane) * _TAILW + tcol
                    tvals = plsc.load_gather(tail, [tidx])
                    stage[pl.ds(sr * NUM_FACTOR + fbg * _L, _L)] = tvals

            pltpu.async_copy(
                stage.at[pl.ds(sr * NUM_FACTOR, NUM_FACTOR)],
                emb_hbm.at[pl.ds(posvec[l] * NUM_FACTOR, NUM_FACTOR)],
                sem_r)
        return t

    lax.fori_loop(0, _RPW // _L, block, 0)
    for _ in range(5):
        drain_win()
    for _ in range(_L):
        pltpu.make_async_copy(
            emb_hbm.at[pl.ds(0, NUM_FACTOR)],
            stage.at[pl.ds(0, NUM_FACTOR)], sem_r).wait()


def _combine_body(uemb_hbm, iemb_hbm, w_hbm, out_hbm,
                  u_v, i_v, w_v, out_v, sem):
    wid = lax.axis_index("s") * _NC + lax.axis_index("c")
    base = wid * _RPW * NUM_FACTOR

    pltpu.sync_copy(uemb_hbm.at[pl.ds(base, _RPW * NUM_FACTOR)], u_v)
    pltpu.sync_copy(iemb_hbm.at[pl.ds(base, _RPW * NUM_FACTOR)], i_v)
    pltpu.sync_copy(w_hbm, w_v)

    w0 = w_v[pl.ds(0, _L)]
    w1 = w_v[pl.ds(_L, _L)]
    w2 = w_v[pl.ds(2 * _L, _L)]
    w3 = w_v[pl.ds(3 * _L, _L)]
    lane_ids = lax.iota(jnp.int32, _L)
    onehot = [lane_ids == l for l in range(_L)]
    rot_idx = [(lane_ids + sh) & (_L - 1) for sh in (8, 4, 2, 1)]

    def blk(b, carry):
        acc = jnp.zeros((_L,), jnp.float32)
        for l in range(_L):
            o = (b * _L + l) * NUM_FACTOR
            v = (u_v[pl.ds(o, _L)] * i_v[pl.ds(o, _L)] * w0
                 + u_v[pl.ds(o + _L, _L)] * i_v[pl.ds(o + _L, _L)] * w1
                 + u_v[pl.ds(o + 2 * _L, _L)] * i_v[pl.ds(o + 2 * _L, _L)] * w2
                 + u_v[pl.ds(o + 3 * _L, _L)] * i_v[pl.ds(o + 3 * _L, _L)] * w3)
            for idx in rot_idx:
                v = v + _rot_gather(v, idx)
            acc = jnp.where(onehot[l], v, acc)
        out_v[pl.ds(b * _L, _L)] = acc
        return carry

    lax.fori_loop(0, _RPW // _L, blk, 0)
    pltpu.sync_copy(out_v, out_hbm.at[pl.ds(wid * _RPW, _RPW)])


@jax.jit
def _gmf(user, item, user_table, item_table, w_flat):
    mesh = plsc.VectorSubcoreMesh(core_axis_name="c", subcore_axis_name="s")
    iota_b = lax.iota(jnp.int32, BATCH)
    us, upos = lax.sort_key_val(user, iota_b)
    its, ipos = lax.sort_key_val(item, iota_b)
    ttu = user_table.T
    tti = item_table.T

    tail_fmt = functools.partial(
        pl.kernel, mesh=mesh,
        out_type=(jax.ShapeDtypeStruct((NUM_FACTOR * _TAILW,), jnp.float32),
                  jax.ShapeDtypeStruct((NUM_FACTOR * _TAILW,), jnp.float32)),
        scratch_types=[
            pltpu.VMEM((NUM_FACTOR, _TAILW), jnp.float32),
            pltpu.VMEM((NUM_FACTOR * _TAILW,), jnp.float32),
            pltpu.SemaphoreType.DMA,
        ],
    )(_tail_fmt_body)
    tlu, tli = tail_fmt(ttu, tti)

    extract = functools.partial(
        pl.kernel, mesh=mesh,
        compiler_params=pltpu.CompilerParams(needs_layout_passes=False),
        out_type=jax.ShapeDtypeStruct((BATCH * NUM_FACTOR,), jnp.float32),
        scratch_types=[
            pltpu.VMEM((_RPW,), jnp.int32),
            pltpu.VMEM((_RPW,), jnp.int32),
            pltpu.VMEM((6, 8, 8, _WINW), jnp.float32),
            pltpu.VMEM((NUM_FACTOR * _TAILW,), jnp.float32),
            pltpu.VMEM((_L * NUM_FACTOR,), jnp.float32),
            pltpu.SemaphoreType.DMA,
            pltpu.SemaphoreType.DMA,
        ],
    )(_extract_body)
    uemb = extract(us, upos, ttu, tlu)
    iemb = extract(its, ipos, tti, tli)

    combine = functools.partial(
        pl.kernel, mesh=mesh,
        out_type=jax.ShapeDtypeStruct((BATCH,), jnp.float32),
        scratch_types=[
            pltpu.VMEM((_RPW * NUM_FACTOR,), jnp.float32),
            pltpu.VMEM((_RPW * NUM_FACTOR,), jnp.float32),
            pltpu.VMEM((NUM_FACTOR,), jnp.float32),
            pltpu.VMEM((_RPW,), jnp.float32),
            pltpu.SemaphoreType.DMA,
        ],
    )(_combine_body)
    return combine(uemb, iemb, w_flat)


def kernel(user, item, user_table, item_table, W):
    return _gmf(user, item, user_table, item_table, W.reshape(-1))


# speedup vs baseline: 4.1608x; 1.0117x over previous
import functools

import jax
import jax.numpy as jnp
from jax import lax
from jax.experimental import pallas as pl
from jax.experimental.pallas import tpu as pltpu
from jax.experimental.pallas import tpu_sc as plsc

NUM_FACTOR = 64
BATCH = 16384

_NC = 2
_NS = 16
_NW = _NC * _NS
_RPW = BATCH // _NW
_L = 16
_WINW = 256
_NROW = 1000000
_TAIL_OFF = (_NROW // 128) * 128
_TAILW = _NROW - _TAIL_OFF
_MAX_WOFF = _TAIL_OFF - _WINW

_GATHER_DNUMS = lax.GatherDimensionNumbers(
    offset_dims=(), collapsed_slice_dims=(0,), start_index_map=(0,))


def _rot_gather(v, idx):
    return lax.gather(v, idx[:, None], _GATHER_DNUMS, slice_sizes=(1,),
                      mode=lax.GatherScatterMode.PROMISE_IN_BOUNDS)


def _win_off(t, base_off):
    return jnp.minimum(base_off + t * _WINW, _MAX_WOFF)


def _tail_fmt_body(ttu_hbm, tti_hbm, outu_hbm, outi_hbm, buf, outv, sem):
    wid = lax.axis_index("s") * _NC + lax.axis_index("c")

    def emit(tt_hbm, out_hbm):
        pltpu.sync_copy(tt_hbm.at[:, pl.ds(_TAIL_OFF, _TAILW)], buf)
        for f in range(NUM_FACTOR):
            for c in range(_TAILW // _L):
                outv[pl.ds(f * _TAILW + c * _L, _L)] = buf[f, pl.ds(c * _L, _L)]
        pltpu.sync_copy(outv, out_hbm)

    @pl.when(wid == 0)
    def _():
        emit(ttu_hbm, outu_hbm)

    @pl.when(wid == 1)
    def _():
        emit(tti_hbm, outi_hbm)


def _extract_body(ids_hbm, pos_hbm, tt_hbm, tlin_hbm, emb_hbm,
                  ids_v, pos_v, chunk, tail, stage, sem_c, sem_r):
    wid = lax.axis_index("s") * _NC + lax.axis_index("c")
    base = wid * _RPW

    pltpu.sync_copy(ids_hbm.at[pl.ds(base, _RPW)], ids_v)
    pltpu.sync_copy(pos_hbm.at[pl.ds(base, _RPW)], pos_v)
    pltpu.sync_copy(tlin_hbm, tail)

    base_off = jnp.minimum(
        lax.shift_right_logical(ids_v[pl.ds(0, _L)][0], 7) * 128, _MAX_WOFF)
    t_cap = (_MAX_WOFF - base_off + _WINW - 1) // _WINW

    def fire(t, slot):
        woff = pl.multiple_of(_win_off(t, base_off), 128)

        def body(fb, carry):
            pltpu.async_copy(
                tt_hbm.at[pl.ds(fb * 8, 8), pl.ds(woff, _WINW)],
                chunk.at[slot, fb], sem_c)
            return carry

        lax.fori_loop(0, 8, body, 0)

    def drain_win():
        def body(fb, carry):
            pltpu.make_async_copy(
                tt_hbm.at[pl.ds(0, 8), pl.ds(0, _WINW)],
                chunk.at[0, 0], sem_c).wait()
            return carry

        lax.fori_loop(0, 8, body, 0)

    for tw in range(7):
        fire(tw, tw)
    drain_win()

    lane = lax.iota(jnp.int32, _L)

    def advance(cb_l, t):
        t_tgt = jnp.maximum(
            t, jnp.minimum((cb_l * 128 - base_off) // _WINW, t_cap))

        def body(s, carry):
            drain_win()
            fire(s + 7, lax.rem(s + 7, 7))
            return carry

        lax.fori_loop(t, t_tgt, body, 0)
        return t_tgt

    def block(b, t):
        idvec = ids_v[pl.ds(b * _L, _L)]
        posvec = pos_v[pl.ds(b * _L, _L)]
        cbvec = lax.shift_right_logical(idvec, 7)
        for l in range(_L):
            t = advance(cbvec[l], t)
            slot = lax.rem(t, 7)
            id_l = idvec[l]
            col = jnp.clip(id_l - _win_off(t, base_off), 0, _WINW - 1)
            cbl = lax.shift_right_logical(col, 7)
            lcol = col & 127
            sr = l
            @pl.when(b > 0)
            def _():
                pltpu.make_async_copy(
                    emb_hbm.at[pl.ds(0, NUM_FACTOR)],
                    stage.at[pl.ds(0, NUM_FACTOR)], sem_r).wait()
            for fbg in range(4):
                f = fbg * _L + lane
                i1 = lax.shift_right_logical(f, 3)
                i2 = f & 7
                i3 = jnp.full((_L,), col, jnp.int32)
                vals = plsc.load_gather(
                    chunk, [jnp.full((_L,), slot, jnp.int32), i1, i2, i3])
                stage[pl.ds(sr * NUM_FACTOR + fbg * _L, _L)] = vals

            @pl.when(id_l >= _TAIL_OFF)
            def _():
                tcol = id_l - _TAIL_OFF
                for fbg in range(4):
                    tidx = (fbg * _L + lane) * _TAILW + tcol
                    tvals = plsc.load_gather(tail, [tidx])
                    stage[pl.ds(sr * NUM_FACTOR + fbg * _L, _L)] = tvals

            pltpu.async_copy(
                stage.at[pl.ds(sr * NUM_FACTOR, NUM_FACTOR)],
                emb_hbm.at[pl.ds(posvec[l] * NUM_FACTOR, NUM_FACTOR)],
                sem_r)
        return t

    lax.fori_loop(0, _RPW // _L, block, 0)
    for _ in range(6):
        drain_win()
    for _ in range(_L):
        pltpu.make_async_copy(
            emb_hbm.at[pl.ds(0, NUM_FACTOR)],
            stage.at[pl.ds(0, NUM_FACTOR)], sem_r).wait()


def _combine_body(uemb_hbm, iemb_hbm, w_hbm, out_hbm,
                  u_v, i_v, w_v, out_v, sem):
    wid = lax.axis_index("s") * _NC + lax.axis_index("c")
    base = wid * _RPW * NUM_FACTOR

    pltpu.sync_copy(uemb_hbm.at[pl.ds(base, _RPW * NUM_FACTOR)], u_v)
    pltpu.sync_copy(iemb_hbm.at[pl.ds(base, _RPW * NUM_FACTOR)], i_v)
    pltpu.sync_copy(w_hbm, w_v)

    w0 = w_v[pl.ds(0, _L)]
    w1 = w_v[pl.ds(_L, _L)]
    w2 = w_v[pl.ds(2 * _L, _L)]
    w3 = w_v[pl.ds(3 * _L, _L)]
    lane_ids = lax.iota(jnp.int32, _L)
    onehot = [lane_ids == l for l in range(_L)]
    rot_idx = [(lane_ids + sh) & (_L - 1) for sh in (8, 4, 2, 1)]

    def blk(b, carry):
        acc = jnp.zeros((_L,), jnp.float32)
        for l in range(_L):
            o = (b * _L + l) * NUM_FACTOR
            v = (u_v[pl.ds(o, _L)] * i_v[pl.ds(o, _L)] * w0
                 + u_v[pl.ds(o + _L, _L)] * i_v[pl.ds(o + _L, _L)] * w1
                 + u_v[pl.ds(o + 2 * _L, _L)] * i_v[pl.ds(o + 2 * _L, _L)] * w2
                 + u_v[pl.ds(o + 3 * _L, _L)] * i_v[pl.ds(o + 3 * _L, _L)] * w3)
            for idx in rot_idx:
                v = v + _rot_gather(v, idx)
            acc = jnp.where(onehot[l], v, acc)
        out_v[pl.ds(b * _L, _L)] = acc
        return carry

    lax.fori_loop(0, _RPW // _L, blk, 0)
    pltpu.sync_copy(out_v, out_hbm.at[pl.ds(wid * _RPW, _RPW)])


@jax.jit
def _gmf(user, item, user_table, item_table, w_flat):
    mesh = plsc.VectorSubcoreMesh(core_axis_name="c", subcore_axis_name="s")
    iota_b = lax.iota(jnp.int32, BATCH)
    us, upos = lax.sort_key_val(user, iota_b)
    its, ipos = lax.sort_key_val(item, iota_b)
    ttu = user_table.T
    tti = item_table.T

    tail_fmt = functools.partial(
        pl.kernel, mesh=mesh,
        out_type=(jax.ShapeDtypeStruct((NUM_FACTOR * _TAILW,), jnp.float32),
                  jax.ShapeDtypeStruct((NUM_FACTOR * _TAILW,), jnp.float32)),
        scratch_types=[
            pltpu.VMEM((NUM_FACTOR, _TAILW), jnp.float32),
            pltpu.VMEM((NUM_FACTOR * _TAILW,), jnp.float32),
            pltpu.SemaphoreType.DMA,
        ],
    )(_tail_fmt_body)
    tlu, tli = tail_fmt(ttu, tti)

    extract = functools.partial(
        pl.kernel, mesh=mesh,
        compiler_params=pltpu.CompilerParams(needs_layout_passes=False),
        out_type=jax.ShapeDtypeStruct((BATCH * NUM_FACTOR,), jnp.float32),
        scratch_types=[
            pltpu.VMEM((_RPW,), jnp.int32),
            pltpu.VMEM((_RPW,), jnp.int32),
            pltpu.VMEM((7, 8, 8, _WINW), jnp.float32),
            pltpu.VMEM((NUM_FACTOR * _TAILW,), jnp.float32),
            pltpu.VMEM((_L * NUM_FACTOR,), jnp.float32),
            pltpu.SemaphoreType.DMA,
            pltpu.SemaphoreType.DMA,
        ],
    )(_extract_body)
    uemb = extract(us, upos, ttu, tlu)
    iemb = extract(its, ipos, tti, tli)

    combine = functools.partial(
        pl.kernel, mesh=mesh,
        out_type=jax.ShapeDtypeStruct((BATCH,), jnp.float32),
        scratch_types=[
            pltpu.VMEM((_RPW * NUM_FACTOR,), jnp.float32),
            pltpu.VMEM((_RPW * NUM_FACTOR,), jnp.float32),
            pltpu.VMEM((NUM_FACTOR,), jnp.float32),
            pltpu.VMEM((_RPW,), jnp.float32),
            pltpu.SemaphoreType.DMA,
        ],
    )(_combine_body)
    return combine(uemb, iemb, w_flat)


def kernel(user, item, user_table, item_table, W):
    return _gmf(user, item, user_table, item_table, W.reshape(-1))
